# Initial kernel scaffold; baseline (speedup 1.0000x reference)
import jax
import jax.numpy as jnp
from jax.experimental import pallas as pl


def kernel(x, table, W, b):
    raise NotImplementedError("write your pallas kernel here")


# speedup vs baseline: 61.0748x; 61.0748x over previous
import jax
import jax.numpy as jnp
from jax import lax
from jax.experimental import pallas as pl
from jax.experimental.pallas import tpu as pltpu
from jax.experimental.pallas import tpu_sc as plsc

B = 4096
G = 4
L = 50
D = 128
V = 10000
NW = 32
ROWS_PER_W = B // NW
IDX_PER_W = ROWS_PER_W * G * L


def _tc_project(w_ref, table_ref, out_ref):
    out_ref[...] = lax.dot_general(
        w_ref[...], table_ref[...],
        (((1,), (1,)), ((), ())),
        preferred_element_type=jnp.float32,
    ) * (1.0 / L)


def _sc_pool(idx_hbm, t2_hbm, bias_hbm, out_hbm, idx_v, t2_v, bias_v, out_v):
    wid = lax.axis_index("s") * 2 + lax.axis_index("c")
    base_row = wid * ROWS_PER_W
    pltpu.sync_copy(idx_hbm.at[pl.ds(base_row * (G * L), IDX_PER_W)], idx_v)
    pltpu.sync_copy(t2_hbm, t2_v)
    pltpu.sync_copy(bias_hbm, bias_v)
    lanes = lax.iota(jnp.int32, 16)
    for rg in range(ROWS_PER_W // 16):
        colbase = lanes * (G * L) + rg * (16 * G * L)
        acc = jnp.zeros((16,), jnp.float32)
        for g in range(G):
            off = g * V

            def body(l, acc, colbase=colbase, off=off, g=g):
                col = colbase + (g * L + l)
                iv = plsc.load_gather(idx_v, [col])
                vals = plsc.load_gather(t2_v, [iv + off])
                return acc + vals

            acc = lax.fori_loop(0, L, body, acc)
        out_v[pl.ds(rg * 16, 16)] = acc + bias_v[...]
    pltpu.sync_copy(out_v, out_hbm.at[pl.ds(base_row, ROWS_PER_W)])


def kernel(x, table, W, b):
    w2 = W.reshape(G, D)
    t2 = pl.pallas_call(
        _tc_project,
        out_shape=jax.ShapeDtypeStruct((G, V), jnp.float32),
    )(w2, table)

    sc = pl.kernel(
        _sc_pool,
        out_type=jax.ShapeDtypeStruct((B,), jnp.float32),
        mesh=plsc.VectorSubcoreMesh(core_axis_name="c", subcore_axis_name="s"),
        compiler_params=pltpu.CompilerParams(needs_layout_passes=False),
        scratch_types=[
            pltpu.VMEM((IDX_PER_W,), jnp.int32),
            pltpu.VMEM((G * V,), jnp.float32),
            pltpu.VMEM((16,), jnp.float32),
            pltpu.VMEM((ROWS_PER_W,), jnp.float32),
        ],
    )
    out = sc(x.reshape(-1).astype(jnp.int32), t2.reshape(-1),
             jnp.broadcast_to(b, (16,)))
    return out.reshape(B, 1)


# speedup vs baseline: 71.1640x; 1.1652x over previous
import jax
import jax.numpy as jnp
from jax import lax
from jax.experimental import pallas as pl
from jax.experimental.pallas import tpu as pltpu
from jax.experimental.pallas import tpu_sc as plsc

B = 4096
G = 4
L = 50
D = 128
V = 10000
NW = 32
ROWS_PER_W = B // NW
IDX_PER_W = ROWS_PER_W * G * L
PAIR = 2 * G * L
NVEC = PAIR // 16


def _tc_project(b_ref, w_ref, table_ref, out_ref):
    out_ref[...] = lax.dot_general(
        w_ref[...], table_ref[...],
        (((1,), (1,)), ((), ())),
        preferred_element_type=jnp.float32,
    ) * (1.0 / L) + b_ref[0] * (1.0 / (G * L))


def _sc_pool(idx_hbm, t2_hbm, out_hbm, idx_v, t2_v, gpat_v, out_v,
             sem_a, sem_b):
    wid = lax.axis_index("s") * 2 + lax.axis_index("c")
    base_row = wid * ROWS_PER_W
    cp_idx = pltpu.async_copy(
        idx_hbm.at[pl.ds(base_row * (G * L), IDX_PER_W)], idx_v, sem_a)
    cp_t2 = pltpu.async_copy(t2_hbm, t2_v, sem_b)
    lanes = lax.iota(jnp.int32, 16)
    half = lanes < 8
    cp_idx.wait()
    cp_t2.wait()
    for k in range(NVEC):
        elem = lanes + 16 * k
        gpat_v[pl.ds(16 * k, 16)] = (elem // L) & (G - 1)

    def rg_body(rg, _):
        outvec = jnp.zeros((16,), jnp.float32)
        for q in range(8):
            base = rg * (8 * PAIR) + q * PAIR
            acc_a = jnp.zeros((16,), jnp.float32)
            acc_b = jnp.zeros((16,), jnp.float32)
            for k in range(NVEC):
                iv = idx_v[pl.ds(base + 16 * k, 16)]
                gv = gpat_v[pl.ds(16 * k, 16)]
                vals = plsc.load_gather(t2_v, [gv, iv])
                if k < NVEC // 2:
                    acc_a = acc_a + vals
                elif k > NVEC // 2:
                    acc_b = acc_b + vals
                else:
                    acc_a = acc_a + jnp.where(half, vals, 0.0)
                    acc_b = acc_b + jnp.where(half, 0.0, vals)
            outvec = outvec + jnp.where(lanes == 2 * q, jnp.sum(acc_a), 0.0)
            outvec = outvec + jnp.where(lanes == 2 * q + 1, jnp.sum(acc_b), 0.0)
        out_v[pl.ds(rg * 16, 16)] = outvec
        return 0

    lax.fori_loop(0, ROWS_PER_W // 16, rg_body, 0)
    pltpu.sync_copy(out_v, out_hbm.at[pl.ds(base_row, ROWS_PER_W)])


def kernel(x, table, W, b):
    w2 = W.reshape(G, D)
    t2 = pl.pallas_call(
        _tc_project,
        in_specs=[
            pl.BlockSpec(memory_space=pltpu.SMEM),
            pl.BlockSpec(memory_space=pltpu.VMEM),
            pl.BlockSpec(memory_space=pltpu.VMEM),
        ],
        out_shape=jax.ShapeDtypeStruct((G, V), jnp.float32),
    )(b, w2, table)

    sc = pl.kernel(
        _sc_pool,
        out_type=jax.ShapeDtypeStruct((B,), jnp.float32),
        mesh=plsc.VectorSubcoreMesh(core_axis_name="c", subcore_axis_name="s"),
        compiler_params=pltpu.CompilerParams(needs_layout_passes=False),
        scratch_types=[
            pltpu.VMEM((IDX_PER_W,), jnp.int32),
            pltpu.VMEM((G, V), jnp.float32),
            pltpu.VMEM((PAIR,), jnp.int32),
            pltpu.VMEM((ROWS_PER_W,), jnp.float32),
            pltpu.SemaphoreType.DMA,
            pltpu.SemaphoreType.DMA,
        ],
    )
    out = sc(x.reshape(-1).astype(jnp.int32), t2)
    return out.reshape(B, 1)


# speedup vs baseline: 73.7811x; 1.0368x over previous
import jax
import jax.numpy as jnp
from jax import lax
from jax.experimental import pallas as pl
from jax.experimental.pallas import tpu as pltpu
from jax.experimental.pallas import tpu_sc as plsc

B = 4096
G = 4
L = 50
D = 128
V = 10000
NW = 32
ROWS_PER_W = B // NW
IDX_PER_W = ROWS_PER_W * G * L
PAIR = 2 * G * L
NVEC = PAIR // 16


def _tc_project(b_ref, w_ref, table_ref, out_ref):
    out_ref[...] = lax.dot_general(
        w_ref[...], table_ref[...],
        (((1,), (1,)), ((), ())),
        preferred_element_type=jnp.float32,
    ) * (1.0 / L) + b_ref[0] * (1.0 / (G * L))


def _sc_pool(idx_hbm, t2_hbm, out_hbm, idx_v, t2_v, gpat_v, out_v,
             sem_a, sem_b):
    wid = lax.axis_index("s") * 2 + lax.axis_index("c")
    base_row = wid * ROWS_PER_W
    cp_idx = pltpu.async_copy(idx_hbm.at[wid], idx_v, sem_a)
    cp_t2 = pltpu.async_copy(t2_hbm, t2_v, sem_b)
    lanes = lax.iota(jnp.int32, 16)
    half = lanes < 8
    cp_idx.wait()
    cp_t2.wait()
    for k in range(NVEC):
        elem = lanes + 16 * k
        gpat_v[pl.ds(16 * k, 16)] = (elem // L) & (G - 1)

    def rg_body(rg, _):
        outvec = jnp.zeros((16,), jnp.float32)
        for q in range(8):
            base = rg * (8 * PAIR) + q * PAIR
            acc_a = jnp.zeros((16,), jnp.float32)
            acc_b = jnp.zeros((16,), jnp.float32)
            for k in range(NVEC):
                iv = idx_v[pl.ds(base + 16 * k, 16)]
                gv = gpat_v[pl.ds(16 * k, 16)]
                vals = plsc.load_gather(t2_v, [gv, iv])
                if k < NVEC // 2:
                    acc_a = acc_a + vals
                elif k > NVEC // 2:
                    acc_b = acc_b + vals
                else:
                    acc_a = acc_a + jnp.where(half, vals, 0.0)
                    acc_b = acc_b + jnp.where(half, 0.0, vals)
            outvec = outvec + jnp.where(lanes == 2 * q, jnp.sum(acc_a), 0.0)
            outvec = outvec + jnp.where(lanes == 2 * q + 1, jnp.sum(acc_b), 0.0)
        out_v[pl.ds(rg * 16, 16)] = outvec
        return 0

    lax.fori_loop(0, ROWS_PER_W // 16, rg_body, 0)
    pltpu.sync_copy(out_v, out_hbm.at[pl.ds(base_row, ROWS_PER_W)])


def kernel(x, table, W, b):
    w2 = W.reshape(G, D)
    t2 = pl.pallas_call(
        _tc_project,
        in_specs=[
            pl.BlockSpec(memory_space=pltpu.SMEM),
            pl.BlockSpec(memory_space=pltpu.VMEM),
            pl.BlockSpec(memory_space=pltpu.VMEM),
        ],
        out_shape=jax.ShapeDtypeStruct((G, V), jnp.float32),
    )(b, w2, table)

    sc = pl.kernel(
        _sc_pool,
        out_type=jax.ShapeDtypeStruct((B,), jnp.float32),
        mesh=plsc.VectorSubcoreMesh(core_axis_name="c", subcore_axis_name="s"),
        compiler_params=pltpu.CompilerParams(needs_layout_passes=False),
        scratch_types=[
            pltpu.VMEM((IDX_PER_W,), jnp.int32),
            pltpu.VMEM((G, V), jnp.float32),
            pltpu.VMEM((PAIR,), jnp.int32),
            pltpu.VMEM((ROWS_PER_W,), jnp.float32),
            pltpu.SemaphoreType.DMA,
            pltpu.SemaphoreType.DMA,
        ],
    )
    x2 = x.astype(jnp.int32).reshape(NW, IDX_PER_W)
    out = sc(x2, t2)
    return out.reshape(B, 1)
